# baseline (speedup 1.0000x reference)
import jax
import jax.numpy as jnp
from jax import lax
from jax.experimental import pallas as pl
from jax.experimental.pallas import tpu as pltpu
import jax.experimental.pallas.tpu_sc as plsc

BATCH = 16384
EMBED = 32
NC = 2
NS = 16
NW = NC * NS
B_PER_W = BATCH // NW
CHUNK = 128
NCHUNK = B_PER_W // CHUNK


def _gather_body(uidx_hbm, iidx_hbm, utab_hbm, itab_hbm, uv_hbm, iv_hbm,
                 uidx_v, iidx_v, urows_v, irows_v, sem):
    w = lax.axis_index("s") * NC + lax.axis_index("c")
    base = w * B_PER_W
    pltpu.sync_copy(uidx_hbm.at[pl.ds(w * NCHUNK, NCHUNK)], uidx_v)
    pltpu.sync_copy(iidx_hbm.at[pl.ds(w * NCHUNK, NCHUNK)], iidx_v)
    copies = []
    for j in range(NCHUNK):
        copies.append(pltpu.async_copy(
            utab_hbm.at[uidx_v.at[j]], urows_v.at[pl.ds(j * CHUNK, CHUNK)], sem))
        copies.append(pltpu.async_copy(
            itab_hbm.at[iidx_v.at[j]], irows_v.at[pl.ds(j * CHUNK, CHUNK)], sem))
    for c in copies:
        c.wait()
    pltpu.sync_copy(urows_v, uv_hbm.at[pl.ds(base, B_PER_W)])
    pltpu.sync_copy(irows_v, iv_hbm.at[pl.ds(base, B_PER_W)])


def _sc_gather(uidx2d, iidx2d, user_table, item_table):
    mesh = plsc.VectorSubcoreMesh(core_axis_name="c", subcore_axis_name="s")
    return pl.kernel(
        _gather_body,
        out_type=(
            jax.ShapeDtypeStruct((BATCH, EMBED), jnp.float32),
            jax.ShapeDtypeStruct((BATCH, EMBED), jnp.float32),
        ),
        mesh=mesh,
        scratch_types=[
            pltpu.VMEM((NCHUNK, CHUNK), jnp.int32),
            pltpu.VMEM((NCHUNK, CHUNK), jnp.int32),
            pltpu.VMEM((B_PER_W, EMBED), jnp.float32),
            pltpu.VMEM((B_PER_W, EMBED), jnp.float32),
            pltpu.SemaphoreType.DMA,
        ],
        compiler_params=pltpu.CompilerParams(use_tc_tiling_on_sc=False),
    )(uidx2d, iidx2d, user_table, item_table)


def _mlp_body(uv_ref, iv_ref, w1u_ref, w1i_ref, b1_ref, w2_ref, b2_ref,
              w3_ref, b3_ref, wo_ref, bo_ref, out_ref):
    h = jnp.dot(uv_ref[...], w1u_ref[...], preferred_element_type=jnp.float32)
    h = h + jnp.dot(iv_ref[...], w1i_ref[...], preferred_element_type=jnp.float32)
    h = jnp.maximum(h + b1_ref[...], 0.0)
    h = jnp.maximum(
        jnp.dot(h, w2_ref[...], preferred_element_type=jnp.float32) + b2_ref[...], 0.0)
    h = jnp.maximum(
        jnp.dot(h, w3_ref[...], preferred_element_type=jnp.float32) + b3_ref[...], 0.0)
    o = jnp.dot(h, wo_ref[...], preferred_element_type=jnp.float32) + bo_ref[...]
    out_ref[...] = jax.nn.sigmoid(o)


def _tc_mlp(uv, iv, W1, b1, W2, b2, W3, b3, Wo, bo):
    BB = 2048
    grid = (BATCH // BB,)
    w1u = W1[:EMBED]
    w1i = W1[EMBED:]
    full = lambda i: (0, 0)
    return pl.pallas_call(
        _mlp_body,
        grid=grid,
        in_specs=[
            pl.BlockSpec((BB, EMBED), lambda i: (i, 0)),
            pl.BlockSpec((BB, EMBED), lambda i: (i, 0)),
            pl.BlockSpec((EMBED, 128), full),
            pl.BlockSpec((EMBED, 128), full),
            pl.BlockSpec((1, 128), full),
            pl.BlockSpec((128, 64), full),
            pl.BlockSpec((1, 64), full),
            pl.BlockSpec((64, 32), full),
            pl.BlockSpec((1, 32), full),
            pl.BlockSpec((32, 1), full),
            pl.BlockSpec((1, 1), full),
        ],
        out_specs=pl.BlockSpec((BB, 1), lambda i: (i, 0)),
        out_shape=jax.ShapeDtypeStruct((BATCH, 1), jnp.float32),
    )(uv, iv, w1u, w1i, b1.reshape(1, 128), W2, b2.reshape(1, 64),
      W3, b3.reshape(1, 32), Wo, bo.reshape(1, 1))


def kernel(user_indices, item_indices, user_table, item_table,
           W1, b1, W2, b2, W3, b3, Wo, bo):
    uidx2d = user_indices.astype(jnp.int32).reshape(NW * NCHUNK, CHUNK)
    iidx2d = item_indices.astype(jnp.int32).reshape(NW * NCHUNK, CHUNK)
    uv, iv = _sc_gather(uidx2d, iidx2d, user_table, item_table)
    return _tc_mlp(uv, iv, W1, b1, W2, b2, W3, b3, Wo, bo)


# speedup vs baseline: 1.4901x; 1.4901x over previous
import jax
import jax.numpy as jnp
from jax import lax
from jax.experimental import pallas as pl
from jax.experimental.pallas import tpu as pltpu
import jax.experimental.pallas.tpu_sc as plsc

BATCH = 16384
EMBED = 32
NC = 2
NS = 16
NW = NC * NS
B_PER_W = BATCH // NW


def _gather_body(uidx_hbm, iidx_hbm, utab_hbm, itab_hbm, uv_hbm, iv_hbm,
                 idx_v, rows_v, sem):
    w = lax.axis_index("s") * NC + lax.axis_index("c")
    base = pl.multiple_of(w * B_PER_W, B_PER_W)

    for idx_hbm, tab_hbm, o_hbm in ((uidx_hbm, utab_hbm, uv_hbm),
                                    (iidx_hbm, itab_hbm, iv_hbm)):
        pltpu.sync_copy(idx_hbm.at[pl.ds(base, B_PER_W)], idx_v)

        def group_step(g, carry, tab_hbm=tab_hbm):
            iv = idx_v[pl.ds(g * 16, 16)]
            for j in range(16):
                r = iv[j]
                pltpu.async_copy(tab_hbm.at[pl.ds(r, 1)],
                                 rows_v.at[pl.ds(g * 16 + j, 1)], sem)
            return carry

        lax.fori_loop(0, B_PER_W // 16, group_step, 0)
        pltpu.make_async_copy(tab_hbm.at[pl.ds(0, B_PER_W)], rows_v, sem).wait()
        pltpu.sync_copy(rows_v, o_hbm.at[pl.ds(base, B_PER_W)])


def _sc_gather(user_indices, item_indices, user_table, item_table):
    mesh = plsc.VectorSubcoreMesh(core_axis_name="c", subcore_axis_name="s")
    return pl.kernel(
        _gather_body,
        out_type=(
            jax.ShapeDtypeStruct((BATCH, EMBED), jnp.float32),
            jax.ShapeDtypeStruct((BATCH, EMBED), jnp.float32),
        ),
        mesh=mesh,
        scratch_types=[
            pltpu.VMEM((B_PER_W,), jnp.int32),
            pltpu.VMEM((B_PER_W, EMBED), jnp.float32),
            pltpu.SemaphoreType.DMA,
        ],
        compiler_params=pltpu.CompilerParams(needs_layout_passes=False),
    )(user_indices, item_indices, user_table, item_table)


def _mlp_body(uv_ref, iv_ref, w1u_ref, w1i_ref, b1_ref, w2_ref, b2_ref,
              w3_ref, b3_ref, wo_ref, bo_ref, out_ref):
    h = jnp.dot(uv_ref[...], w1u_ref[...], preferred_element_type=jnp.float32)
    h = h + jnp.dot(iv_ref[...], w1i_ref[...], preferred_element_type=jnp.float32)
    h = jnp.maximum(h + b1_ref[...], 0.0)
    h = jnp.maximum(
        jnp.dot(h, w2_ref[...], preferred_element_type=jnp.float32) + b2_ref[...], 0.0)
    h = jnp.maximum(
        jnp.dot(h, w3_ref[...], preferred_element_type=jnp.float32) + b3_ref[...], 0.0)
    o = jnp.dot(h, wo_ref[...], preferred_element_type=jnp.float32) + bo_ref[...]
    out_ref[...] = jax.nn.sigmoid(o)


def _tc_mlp(uv, iv, W1, b1, W2, b2, W3, b3, Wo, bo):
    BB = 2048
    grid = (BATCH // BB,)
    w1u = W1[:EMBED]
    w1i = W1[EMBED:]
    full = lambda i: (0, 0)
    return pl.pallas_call(
        _mlp_body,
        grid=grid,
        in_specs=[
            pl.BlockSpec((BB, EMBED), lambda i: (i, 0)),
            pl.BlockSpec((BB, EMBED), lambda i: (i, 0)),
            pl.BlockSpec((EMBED, 128), full),
            pl.BlockSpec((EMBED, 128), full),
            pl.BlockSpec((1, 128), full),
            pl.BlockSpec((128, 64), full),
            pl.BlockSpec((1, 64), full),
            pl.BlockSpec((64, 32), full),
            pl.BlockSpec((1, 32), full),
            pl.BlockSpec((32, 1), full),
            pl.BlockSpec((1, 1), full),
        ],
        out_specs=pl.BlockSpec((BB, 1), lambda i: (i, 0)),
        out_shape=jax.ShapeDtypeStruct((BATCH, 1), jnp.float32),
    )(uv, iv, w1u, w1i, b1.reshape(1, 128), W2, b2.reshape(1, 64),
      W3, b3.reshape(1, 32), Wo, bo.reshape(1, 1))


def kernel(user_indices, item_indices, user_table, item_table,
           W1, b1, W2, b2, W3, b3, Wo, bo):
    uv, iv = _sc_gather(user_indices.astype(jnp.int32),
                        item_indices.astype(jnp.int32),
                        user_table, item_table)
    return _tc_mlp(uv, iv, W1, b1, W2, b2, W3, b3, Wo, bo)
